# baseline (speedup 1.0000x reference)
import functools

import jax
import jax.numpy as jnp
from jax import lax
from jax.experimental import pallas as pl
from jax.experimental.pallas import tpu as pltpu
from jax.experimental.pallas import tpu_sc as plsc

N = 100000
C = 256
G = 256
H = 16

NC = 2
NS = 16
NW = NC * NS

K = 64
NCHUNK = (N + K - 1) // K
TRASH = G
GP = 272
TPW = (NCHUNK + NW - 1) // NW

BLK = 2000
NBLK = N // BLK

_mesh = plsc.VectorSubcoreMesh(core_axis_name="c", subcore_axis_name="s")

GPV = 264


@functools.partial(
    pl.kernel,
    mesh=_mesh,
    compiler_params=pltpu.CompilerParams(needs_layout_passes=False),
    out_type=[
        jax.ShapeDtypeStruct((NW, G * C), jnp.float32),
        jax.ShapeDtypeStruct((NW, G * 16), jnp.float32),
    ],
    scratch_types=[
        pltpu.VMEM((K, C), jnp.float32),
        pltpu.VMEM((K, 16), jnp.int32),
        pltpu.VMEM((K,), jnp.int32),
        pltpu.VMEM((GPV * C,), jnp.float32),
        pltpu.VMEM((GPV * 16,), jnp.float32),
    ],
)
def _segsum_sc(x_hbm, av_hbm, ca_hbm, zc_hbm, z16_hbm,
               acc_out, cnt_out, xv, av, ca, acc_f, cnt_f):
    cid = lax.axis_index("c")
    sid = lax.axis_index("s")
    wid = sid * NC + cid

    pltpu.sync_copy(zc_hbm, acc_f)
    pltpu.sync_copy(z16_hbm, cnt_f)
    ones16 = jnp.ones((16,), jnp.float32)

    def chunk_body(t, _):
        chunk = wid * TPW + t

        @pl.when(chunk < NCHUNK)
        def _do():
            start = jnp.minimum(chunk * K, N - K)
            pltpu.sync_copy(av_hbm.at[chunk], av)
            pltpu.sync_copy(ca_hbm.at[chunk], ca)
            pltpu.sync_copy(x_hbm.at[pl.ds(start, K)], xv)

            for g in range(K // 16):
                plsc.addupdate_scatter(cnt_f, [ca[pl.ds(g * 16, 16)]], ones16)
                for j in range(16):
                    addr = av[g * 16 + j, :]
                    for l in range(C // 16):
                        plsc.addupdate_scatter(
                            acc_f, [addr + l * 16],
                            xv[g * 16 + j, pl.ds(l * 16, 16)])
        return ()

    lax.fori_loop(0, TPW, chunk_body, ())

    pltpu.sync_copy(acc_f.at[pl.ds(0, G * C)], acc_out.at[wid])
    pltpu.sync_copy(cnt_f.at[pl.ds(0, G * 16)], cnt_out.at[wid])


def _mlp_body(acc_ref, cnt_ref, w1_ref, a_ref, w2_ref, s_ref):
    seg = jnp.sum(acc_ref[...], axis=0)
    cnt_full = jnp.sum(cnt_ref[...], axis=0)
    cnt = jnp.maximum(jnp.sum(cnt_full, axis=1, keepdims=True), 1.0)
    x_avg = seg / cnt
    h = lax.dot_general(x_avg, w1_ref[...], (((1,), (1,)), ((), ())),
                        preferred_element_type=jnp.float32)
    a = a_ref[0]
    h = jnp.where(h >= 0, h, a * h)
    h = lax.dot_general(h, w2_ref[...], (((1,), (1,)), ((), ())),
                        preferred_element_type=jnp.float32)
    s_ref[...] = jax.nn.sigmoid(h)


def _scale_body(x_ref, b_ref, s_ref, o_ref):
    b = b_ref[0, 0, :]
    gi = lax.broadcasted_iota(jnp.int32, (BLK, G), 1)
    oh = jnp.where(gi == b[:, None], 1.0, 0.0).astype(jnp.float32)
    se = lax.dot_general(oh, s_ref[...], (((1,), (0,)), ((), ())),
                         preferred_element_type=jnp.float32)
    o_ref[...] = x_ref[...] * se


def kernel(x, batch, num_graphs, W1, a, W2):
    bi = jnp.minimum(batch, num_graphs - 1).astype(jnp.int32)

    nfull = N // K
    main = bi[: nfull * K].reshape(nfull, K)
    tail = bi[N - K:].reshape(1, K)
    tail_fresh = (jnp.arange(K) >= (nfull * K - (N - K)))[None, :]
    tail = jnp.where(tail_fresh, tail, TRASH)
    bidx = jnp.concatenate([main, tail], axis=0).astype(jnp.int32)

    lane16 = jnp.arange(16, dtype=jnp.int32)
    av_tab = bidx[:, :, None] * C + lane16[None, None, :]
    ca_tab = bidx * 16 + jnp.tile(lane16, K // 16)[None, :]

    zc = jnp.zeros((GPV * C,), jnp.float32)
    z16 = jnp.zeros((GPV * 16,), jnp.float32)

    acc, cnt = _segsum_sc(x, av_tab, ca_tab, zc, z16)
    acc = acc.reshape(NW, G, C)
    cnt = cnt.reshape(NW, G, 16)

    s = pl.pallas_call(
        _mlp_body,
        in_specs=[
            pl.BlockSpec((NW, G, C), lambda: (0, 0, 0)),
            pl.BlockSpec((NW, G, 16), lambda: (0, 0, 0)),
            pl.BlockSpec((H, C), lambda: (0, 0)),
            pl.BlockSpec(memory_space=pltpu.SMEM),
            pl.BlockSpec((C, H), lambda: (0, 0)),
        ],
        out_specs=pl.BlockSpec((G, C), lambda: (0, 0)),
        out_shape=jax.ShapeDtypeStruct((G, C), jnp.float32),
    )(acc, cnt, W1, a.reshape(1), W2)

    b3 = bi.reshape(NBLK, 1, BLK)
    out = pl.pallas_call(
        _scale_body,
        grid=(NBLK,),
        in_specs=[
            pl.BlockSpec((BLK, C), lambda i: (i, 0)),
            pl.BlockSpec((1, 1, BLK), lambda i: (i, 0, 0)),
            pl.BlockSpec((G, C), lambda i: (0, 0)),
        ],
        out_specs=pl.BlockSpec((BLK, C), lambda i: (i, 0)),
        out_shape=jax.ShapeDtypeStruct((N, C), jnp.float32),
    )(x, b3, s)
    return out


# speedup vs baseline: 1.0613x; 1.0613x over previous
import functools

import jax
import jax.numpy as jnp
from jax import lax
from jax.experimental import pallas as pl
from jax.experimental.pallas import tpu as pltpu
from jax.experimental.pallas import tpu_sc as plsc

N = 100000
C = 256
G = 256
H = 16

NC = 2
NS = 16
NW = NC * NS

K = 64
NCHUNK = (N + K - 1) // K
TRASH = G
GP = 272
TPW = (NCHUNK + NW - 1) // NW

BLK = 2000
NBLK = N // BLK

_mesh = plsc.VectorSubcoreMesh(core_axis_name="c", subcore_axis_name="s")

GPV = 264
PCHUNK = TPW * NW
NL = C // 16


@functools.partial(
    pl.kernel,
    mesh=_mesh,
    compiler_params=pltpu.CompilerParams(needs_layout_passes=False),
    out_type=[
        jax.ShapeDtypeStruct((NW, G * C), jnp.float32),
        jax.ShapeDtypeStruct((NW, G * 16), jnp.float32),
    ],
    scratch_types=[
        pltpu.VMEM((K, C), jnp.float32),
        pltpu.VMEM((K, 16), jnp.int32),
        pltpu.VMEM((K,), jnp.int32),
        pltpu.VMEM((GPV * C,), jnp.float32),
        pltpu.VMEM((GPV * 16,), jnp.float32),
    ],
)
def _segsum_sc(x_hbm, av_hbm, ca_hbm, zc_hbm, z16_hbm,
               acc_out, cnt_out, xv, av, ca, acc_f, cnt_f):
    cid = lax.axis_index("c")
    sid = lax.axis_index("s")
    wid = sid * NC + cid

    pltpu.sync_copy(zc_hbm, acc_f)
    pltpu.sync_copy(z16_hbm, cnt_f)
    ones16 = jnp.ones((16,), jnp.float32)

    zero16 = jnp.zeros((16,), jnp.float32)
    prev0 = jnp.full((16,), TRASH * C, jnp.int32) + lax.iota(jnp.int32, 16)

    def chunk_body(t, carry):
        prev = carry[0]
        runs = list(carry[1:])
        chunk = wid * TPW + t
        start = jnp.minimum(chunk * K, N - K)
        pltpu.sync_copy(av_hbm.at[chunk], av)
        pltpu.sync_copy(ca_hbm.at[chunk], ca)
        pltpu.sync_copy(x_hbm.at[pl.ds(start, K)], xv)

        for g in range(K // 16):
            plsc.addupdate_scatter(cnt_f, [ca[pl.ds(g * 16, 16)]], ones16)
        for j in range(K):
            addr = av[j, :]
            neq = jnp.sum(jnp.where(addr != prev, 1, 0), axis=0) != 0
            pv, rv = prev, list(runs)

            @pl.when(neq)
            def _flush():
                for l in range(NL):
                    plsc.addupdate_scatter(acc_f, [pv + l * 16], rv[l])

            m = jnp.broadcast_to(neq, (16,))
            for l in range(NL):
                xr = xv[j, pl.ds(l * 16, 16)]
                runs[l] = jnp.where(m, xr, runs[l] + xr)
            prev = addr
        return (prev, *runs)

    final = lax.fori_loop(0, TPW, chunk_body, (prev0, *([zero16] * NL)))
    fprev = final[0]
    for l in range(NL):
        plsc.addupdate_scatter(acc_f, [fprev + l * 16], final[1 + l])

    pltpu.sync_copy(acc_f.at[pl.ds(0, G * C)], acc_out.at[wid])
    pltpu.sync_copy(cnt_f.at[pl.ds(0, G * 16)], cnt_out.at[wid])


def _mlp_body(acc_ref, cnt_ref, w1_ref, a_ref, w2_ref, s_ref):
    seg = jnp.sum(acc_ref[...], axis=0)
    cnt_full = jnp.sum(cnt_ref[...], axis=0)
    cnt = jnp.maximum(jnp.sum(cnt_full, axis=1, keepdims=True), 1.0)
    x_avg = seg / cnt
    h = lax.dot_general(x_avg, w1_ref[...], (((1,), (1,)), ((), ())),
                        preferred_element_type=jnp.float32)
    a = a_ref[0]
    h = jnp.where(h >= 0, h, a * h)
    h = lax.dot_general(h, w2_ref[...], (((1,), (1,)), ((), ())),
                        preferred_element_type=jnp.float32)
    s_ref[...] = jax.nn.sigmoid(h)


def _scale_body(x_ref, b_ref, s_ref, o_ref):
    b = b_ref[0, 0, :]
    gi = lax.broadcasted_iota(jnp.int32, (BLK, G), 1)
    oh = jnp.where(gi == b[:, None], 1.0, 0.0).astype(jnp.float32)
    se = lax.dot_general(oh, s_ref[...], (((1,), (0,)), ((), ())),
                         preferred_element_type=jnp.float32)
    o_ref[...] = x_ref[...] * se


def kernel(x, batch, num_graphs, W1, a, W2):
    bi = jnp.minimum(batch, num_graphs - 1).astype(jnp.int32)

    nfull = N // K
    main = bi[: nfull * K].reshape(nfull, K)
    tail = bi[N - K:].reshape(1, K)
    tail_fresh = (jnp.arange(K) >= (nfull * K - (N - K)))[None, :]
    tail = jnp.where(tail_fresh, tail, TRASH)
    pad = jnp.full((PCHUNK - NCHUNK, K), TRASH, bi.dtype)
    bidx = jnp.concatenate([main, tail, pad], axis=0).astype(jnp.int32)

    lane16 = jnp.arange(16, dtype=jnp.int32)
    av_tab = bidx[:, :, None] * C + lane16[None, None, :]
    ca_tab = bidx * 16 + jnp.tile(lane16, K // 16)[None, :]

    zc = jnp.zeros((GPV * C,), jnp.float32)
    z16 = jnp.zeros((GPV * 16,), jnp.float32)

    acc, cnt = _segsum_sc(x, av_tab, ca_tab, zc, z16)
    acc = acc.reshape(NW, G, C)
    cnt = cnt.reshape(NW, G, 16)

    s = pl.pallas_call(
        _mlp_body,
        in_specs=[
            pl.BlockSpec((NW, G, C), lambda: (0, 0, 0)),
            pl.BlockSpec((NW, G, 16), lambda: (0, 0, 0)),
            pl.BlockSpec((H, C), lambda: (0, 0)),
            pl.BlockSpec(memory_space=pltpu.SMEM),
            pl.BlockSpec((C, H), lambda: (0, 0)),
        ],
        out_specs=pl.BlockSpec((G, C), lambda: (0, 0)),
        out_shape=jax.ShapeDtypeStruct((G, C), jnp.float32),
    )(acc, cnt, W1, a.reshape(1), W2)

    b3 = bi.reshape(NBLK, 1, BLK)
    out = pl.pallas_call(
        _scale_body,
        grid=(NBLK,),
        in_specs=[
            pl.BlockSpec((BLK, C), lambda i: (i, 0)),
            pl.BlockSpec((1, 1, BLK), lambda i: (i, 0, 0)),
            pl.BlockSpec((G, C), lambda i: (0, 0)),
        ],
        out_specs=pl.BlockSpec((BLK, C), lambda i: (i, 0)),
        out_shape=jax.ShapeDtypeStruct((N, C), jnp.float32),
    )(x, b3, s)
    return out


# speedup vs baseline: 1.2407x; 1.1691x over previous
import functools

import jax
import jax.numpy as jnp
from jax import lax
from jax.experimental import pallas as pl
from jax.experimental.pallas import tpu as pltpu
from jax.experimental.pallas import tpu_sc as plsc

N = 100000
C = 256
G = 256
H = 16

NC = 2
NS = 16
NW = NC * NS

K = 64
NCHUNK = (N + K - 1) // K
TRASH = G
GP = 272
TPW = (NCHUNK + NW - 1) // NW

BLK = 2000
NBLK = N // BLK

_mesh = plsc.VectorSubcoreMesh(core_axis_name="c", subcore_axis_name="s")

GPV = 264
TPW2 = (TPW + 1) // 2
TPWE = TPW2 * 2
PCHUNK = TPWE * NW
NL = C // 16
KA = K + K // 16


@functools.partial(
    pl.kernel,
    mesh=_mesh,
    compiler_params=pltpu.CompilerParams(needs_layout_passes=False),
    out_type=[
        jax.ShapeDtypeStruct((NW, G * C), jnp.float32),
        jax.ShapeDtypeStruct((NW, G * 16), jnp.float32),
    ],
    scratch_types=[
        pltpu.VMEM((K, C), jnp.float32),
        pltpu.VMEM((K, C), jnp.float32),
        pltpu.VMEM((KA, 16), jnp.int32),
        pltpu.VMEM((KA, 16), jnp.int32),
        pltpu.VMEM((GPV * C,), jnp.float32),
        pltpu.VMEM((GPV * 16,), jnp.float32),
        pltpu.SemaphoreType.DMA,
        pltpu.SemaphoreType.DMA,
        pltpu.SemaphoreType.DMA,
        pltpu.SemaphoreType.DMA,
    ],
)
def _segsum_sc(x_hbm, av_hbm, zc_hbm, z16_hbm,
               acc_out, cnt_out, xva, xvb, ava, avb, acc_f, cnt_f,
               sxa, sxb, saa, sab):
    cid = lax.axis_index("c")
    sid = lax.axis_index("s")
    wid = sid * NC + cid
    c0 = wid * TPWE

    pltpu.sync_copy(zc_hbm, acc_f)
    pltpu.sync_copy(z16_hbm, cnt_f)
    ones16 = jnp.ones((16,), jnp.float32)
    zero16 = jnp.zeros((16,), jnp.float32)
    prev0 = jnp.full((16,), TRASH * C, jnp.int32) + lax.iota(jnp.int32, 16)

    def start_loads(chunk, xbuf, abuf, sx, sa):
        chunk = jnp.minimum(chunk, PCHUNK - 1)
        start = jnp.minimum(chunk * K, N - K)
        pltpu.async_copy(x_hbm.at[pl.ds(start, K)], xbuf, sx)
        pltpu.async_copy(av_hbm.at[chunk], abuf, sa)

    def wait_loads(xbuf, abuf, sx, sa):
        pltpu.make_async_copy(x_hbm.at[pl.ds(0, K)], xbuf, sx).wait()
        pltpu.make_async_copy(av_hbm.at[0], abuf, sa).wait()

    def compute(xbuf, abuf, carry):
        prev = carry[0]
        runs = list(carry[1:])
        for g in range(K // 16):
            plsc.addupdate_scatter(cnt_f, [abuf[K + g, :]], ones16)
        for j in range(K):
            addr = abuf[j, :]
            neq = jnp.sum(jnp.where(addr != prev, 1, 0), axis=0) != 0
            pv, rv = prev, list(runs)

            @pl.when(neq)
            def _flush():
                for l in range(NL):
                    plsc.addupdate_scatter(acc_f, [pv + l * 16], rv[l])

            m = jnp.broadcast_to(neq, (16,))
            for l in range(NL):
                xr = xbuf[j, pl.ds(l * 16, 16)]
                runs[l] = jnp.where(m, xr, runs[l] + xr)
            prev = addr
        return (prev, *runs)

    start_loads(c0, xva, ava, sxa, saa)

    def pair_body(t2, carry):
        ca = c0 + 2 * t2
        start_loads(ca + 1, xvb, avb, sxb, sab)
        wait_loads(xva, ava, sxa, saa)
        carry = compute(xva, ava, carry)
        start_loads(ca + 2, xva, ava, sxa, saa)
        wait_loads(xvb, avb, sxb, sab)
        carry = compute(xvb, avb, carry)
        return carry

    final = lax.fori_loop(0, TPW2, pair_body, (prev0, *([zero16] * NL)))
    wait_loads(xva, ava, sxa, saa)
    fprev = final[0]
    for l in range(NL):
        plsc.addupdate_scatter(acc_f, [fprev + l * 16], final[1 + l])

    pltpu.sync_copy(acc_f.at[pl.ds(0, G * C)], acc_out.at[wid])
    pltpu.sync_copy(cnt_f.at[pl.ds(0, G * 16)], cnt_out.at[wid])


def _mlp_body(acc_ref, cnt_ref, w1_ref, a_ref, w2_ref, s_ref):
    seg = jnp.sum(acc_ref[...], axis=0)
    cnt_full = jnp.sum(cnt_ref[...], axis=0)
    cnt = jnp.maximum(jnp.sum(cnt_full, axis=1, keepdims=True), 1.0)
    x_avg = seg / cnt
    h = lax.dot_general(x_avg, w1_ref[...], (((1,), (1,)), ((), ())),
                        preferred_element_type=jnp.float32)
    a = a_ref[0]
    h = jnp.where(h >= 0, h, a * h)
    h = lax.dot_general(h, w2_ref[...], (((1,), (1,)), ((), ())),
                        preferred_element_type=jnp.float32)
    s_ref[...] = jax.nn.sigmoid(h)


def _scale_body(x_ref, b_ref, s_ref, o_ref):
    b = b_ref[0, 0, :]
    gi = lax.broadcasted_iota(jnp.int32, (BLK, G), 1)
    oh = jnp.where(gi == b[:, None], 1.0, 0.0).astype(jnp.float32)
    se = lax.dot_general(oh, s_ref[...], (((1,), (0,)), ((), ())),
                         preferred_element_type=jnp.float32)
    o_ref[...] = x_ref[...] * se


def kernel(x, batch, num_graphs, W1, a, W2):
    bi = jnp.minimum(batch, num_graphs - 1).astype(jnp.int32)

    nfull = N // K
    main = bi[: nfull * K].reshape(nfull, K)
    tail = bi[N - K:].reshape(1, K)
    tail_fresh = (jnp.arange(K) >= (nfull * K - (N - K)))[None, :]
    tail = jnp.where(tail_fresh, tail, TRASH)
    pad = jnp.full((PCHUNK - NCHUNK, K), TRASH, bi.dtype)
    bidx = jnp.concatenate([main, tail, pad], axis=0).astype(jnp.int32)

    lane16 = jnp.arange(16, dtype=jnp.int32)
    av_tab = bidx[:, :, None] * C + lane16[None, None, :]
    ca_tab = (bidx * 16 + jnp.tile(lane16, K // 16)[None, :]).reshape(
        PCHUNK, K // 16, 16)
    aux = jnp.concatenate([av_tab, ca_tab], axis=1)

    zc = jnp.zeros((GPV * C,), jnp.float32)
    z16 = jnp.zeros((GPV * 16,), jnp.float32)

    acc, cnt = _segsum_sc(x, aux, zc, z16)
    acc = acc.reshape(NW, G, C)
    cnt = cnt.reshape(NW, G, 16)

    s = pl.pallas_call(
        _mlp_body,
        in_specs=[
            pl.BlockSpec((NW, G, C), lambda: (0, 0, 0)),
            pl.BlockSpec((NW, G, 16), lambda: (0, 0, 0)),
            pl.BlockSpec((H, C), lambda: (0, 0)),
            pl.BlockSpec(memory_space=pltpu.SMEM),
            pl.BlockSpec((C, H), lambda: (0, 0)),
        ],
        out_specs=pl.BlockSpec((G, C), lambda: (0, 0)),
        out_shape=jax.ShapeDtypeStruct((G, C), jnp.float32),
    )(acc, cnt, W1, a.reshape(1), W2)

    b3 = bi.reshape(NBLK, 1, BLK)
    out = pl.pallas_call(
        _scale_body,
        grid=(NBLK,),
        in_specs=[
            pl.BlockSpec((BLK, C), lambda i: (i, 0)),
            pl.BlockSpec((1, 1, BLK), lambda i: (i, 0, 0)),
            pl.BlockSpec((G, C), lambda i: (0, 0)),
        ],
        out_specs=pl.BlockSpec((BLK, C), lambda i: (i, 0)),
        out_shape=jax.ShapeDtypeStruct((N, C), jnp.float32),
    )(x, b3, s)
    return out


# speedup vs baseline: 1.3765x; 1.1095x over previous
import functools

import jax
import jax.numpy as jnp
from jax import lax
from jax.experimental import pallas as pl
from jax.experimental.pallas import tpu as pltpu
from jax.experimental.pallas import tpu_sc as plsc

N = 100000
C = 256
G = 256
H = 16

NC = 2
NS = 16
NW = NC * NS

K = 64
NCHUNK = (N + K - 1) // K
TRASH = G
GP = 272
TPW = (NCHUNK + NW - 1) // NW

BLK = 2000
NBLK = N // BLK

_mesh = plsc.VectorSubcoreMesh(core_axis_name="c", subcore_axis_name="s")

GPV = 264
TPW2 = (TPW + 1) // 2
TPWE = TPW2 * 2
PCHUNK = TPWE * NW
NL = C // 16
KA = K + K // 16


@functools.partial(
    pl.kernel,
    mesh=_mesh,
    compiler_params=pltpu.CompilerParams(needs_layout_passes=False),
    out_type=[
        jax.ShapeDtypeStruct((NW, G * C), jnp.float32),
        jax.ShapeDtypeStruct((NW, G * 16), jnp.float32),
    ],
    scratch_types=[
        pltpu.VMEM((K, C), jnp.float32),
        pltpu.VMEM((K, C), jnp.float32),
        pltpu.VMEM((KA, 16), jnp.int32),
        pltpu.VMEM((KA, 16), jnp.int32),
        pltpu.VMEM((GPV * C,), jnp.float32),
        pltpu.VMEM((GPV * 16,), jnp.float32),
        pltpu.SemaphoreType.DMA,
        pltpu.SemaphoreType.DMA,
        pltpu.SemaphoreType.DMA,
        pltpu.SemaphoreType.DMA,
    ],
)
def _segsum_sc(x_hbm, av_hbm, zc_hbm, z16_hbm,
               acc_out, cnt_out, xva, xvb, ava, avb, acc_f, cnt_f,
               sxa, sxb, saa, sab):
    cid = lax.axis_index("c")
    sid = lax.axis_index("s")
    wid = sid * NC + cid
    c0 = wid * TPWE

    pltpu.sync_copy(zc_hbm, acc_f)
    pltpu.sync_copy(z16_hbm, cnt_f)
    ones16 = jnp.ones((16,), jnp.float32)
    zero16 = jnp.zeros((16,), jnp.float32)
    prev0 = jnp.full((16,), TRASH * C, jnp.int32) + lax.iota(jnp.int32, 16)

    def start_loads(chunk, xbuf, abuf, sx, sa):
        chunk = jnp.minimum(chunk, PCHUNK - 1)
        start = jnp.minimum(chunk * K, N - K)
        pltpu.async_copy(x_hbm.at[pl.ds(start, K)], xbuf, sx)
        pltpu.async_copy(av_hbm.at[chunk], abuf, sa)

    def wait_loads(xbuf, abuf, sx, sa):
        pltpu.make_async_copy(x_hbm.at[pl.ds(0, K)], xbuf, sx).wait()
        pltpu.make_async_copy(av_hbm.at[0], abuf, sa).wait()

    def compute(xbuf, abuf, carry):
        prev = carry[0]
        runs = list(carry[1:])
        for g in range(K // 16):
            plsc.addupdate_scatter(cnt_f, [abuf[K + g, :]], ones16)
        for j in range(K):
            addr = abuf[j, :]
            ndiff = plsc.all_reduce_population_count(addr != prev)
            m = ndiff != 0
            for l in range(NL):
                plsc.addupdate_scatter(acc_f, [prev + l * 16], runs[l], mask=m)
            for l in range(NL):
                xr = xbuf[j, pl.ds(l * 16, 16)]
                runs[l] = jnp.where(m, xr, runs[l] + xr)
            prev = addr
        return (prev, *runs)

    start_loads(c0, xva, ava, sxa, saa)

    def pair_body(t2, carry):
        ca = c0 + 2 * t2
        start_loads(ca + 1, xvb, avb, sxb, sab)
        wait_loads(xva, ava, sxa, saa)
        carry = compute(xva, ava, carry)
        start_loads(ca + 2, xva, ava, sxa, saa)
        wait_loads(xvb, avb, sxb, sab)
        carry = compute(xvb, avb, carry)
        return carry

    final = lax.fori_loop(0, TPW2, pair_body, (prev0, *([zero16] * NL)))
    wait_loads(xva, ava, sxa, saa)
    fprev = final[0]
    for l in range(NL):
        plsc.addupdate_scatter(acc_f, [fprev + l * 16], final[1 + l])

    pltpu.sync_copy(acc_f.at[pl.ds(0, G * C)], acc_out.at[wid])
    pltpu.sync_copy(cnt_f.at[pl.ds(0, G * 16)], cnt_out.at[wid])


def _mlp_body(acc_ref, cnt_ref, w1_ref, a_ref, w2_ref, s_ref):
    seg = jnp.sum(acc_ref[...], axis=0)
    cnt_full = jnp.sum(cnt_ref[...], axis=0)
    cnt = jnp.maximum(jnp.sum(cnt_full, axis=1, keepdims=True), 1.0)
    x_avg = seg / cnt
    h = lax.dot_general(x_avg, w1_ref[...], (((1,), (1,)), ((), ())),
                        preferred_element_type=jnp.float32)
    a = a_ref[0]
    h = jnp.where(h >= 0, h, a * h)
    h = lax.dot_general(h, w2_ref[...], (((1,), (1,)), ((), ())),
                        preferred_element_type=jnp.float32)
    s_ref[...] = jax.nn.sigmoid(h)


def _scale_body(x_ref, b_ref, s_ref, o_ref):
    b = b_ref[0, 0, :]
    gi = lax.broadcasted_iota(jnp.int32, (BLK, G), 1)
    oh = jnp.where(gi == b[:, None], 1.0, 0.0).astype(jnp.float32)
    se = lax.dot_general(oh, s_ref[...], (((1,), (0,)), ((), ())),
                         preferred_element_type=jnp.float32)
    o_ref[...] = x_ref[...] * se


def kernel(x, batch, num_graphs, W1, a, W2):
    bi = jnp.minimum(batch, num_graphs - 1).astype(jnp.int32)

    nfull = N // K
    main = bi[: nfull * K].reshape(nfull, K)
    tail = bi[N - K:].reshape(1, K)
    tail_fresh = (jnp.arange(K) >= (nfull * K - (N - K)))[None, :]
    tail = jnp.where(tail_fresh, tail, TRASH)
    pad = jnp.full((PCHUNK - NCHUNK, K), TRASH, bi.dtype)
    bidx = jnp.concatenate([main, tail, pad], axis=0).astype(jnp.int32)

    lane16 = jnp.arange(16, dtype=jnp.int32)
    av_tab = bidx[:, :, None] * C + lane16[None, None, :]
    ca_tab = (bidx * 16 + jnp.tile(lane16, K // 16)[None, :]).reshape(
        PCHUNK, K // 16, 16)
    aux = jnp.concatenate([av_tab, ca_tab], axis=1)

    zc = jnp.zeros((GPV * C,), jnp.float32)
    z16 = jnp.zeros((GPV * 16,), jnp.float32)

    acc, cnt = _segsum_sc(x, aux, zc, z16)
    acc = acc.reshape(NW, G, C)
    cnt = cnt.reshape(NW, G, 16)

    s = pl.pallas_call(
        _mlp_body,
        in_specs=[
            pl.BlockSpec((NW, G, C), lambda: (0, 0, 0)),
            pl.BlockSpec((NW, G, 16), lambda: (0, 0, 0)),
            pl.BlockSpec((H, C), lambda: (0, 0)),
            pl.BlockSpec(memory_space=pltpu.SMEM),
            pl.BlockSpec((C, H), lambda: (0, 0)),
        ],
        out_specs=pl.BlockSpec((G, C), lambda: (0, 0)),
        out_shape=jax.ShapeDtypeStruct((G, C), jnp.float32),
    )(acc, cnt, W1, a.reshape(1), W2)

    b3 = bi.reshape(NBLK, 1, BLK)
    out = pl.pallas_call(
        _scale_body,
        grid=(NBLK,),
        in_specs=[
            pl.BlockSpec((BLK, C), lambda i: (i, 0)),
            pl.BlockSpec((1, 1, BLK), lambda i: (i, 0, 0)),
            pl.BlockSpec((G, C), lambda i: (0, 0)),
        ],
        out_specs=pl.BlockSpec((BLK, C), lambda i: (i, 0)),
        out_shape=jax.ShapeDtypeStruct((N, C), jnp.float32),
    )(x, b3, s)
    return out


# speedup vs baseline: 1.4507x; 1.0539x over previous
import functools

import jax
import jax.numpy as jnp
from jax import lax
from jax.experimental import pallas as pl
from jax.experimental.pallas import tpu as pltpu
from jax.experimental.pallas import tpu_sc as plsc

N = 100000
C = 256
G = 256
H = 16

NC = 2
NS = 16
NW = NC * NS

K = 32
NCHUNK = (N + K - 1) // K
TRASH = G
GP = 272
TPW = (NCHUNK + NW - 1) // NW

BLK = 2000
NBLK = N // BLK

_mesh = plsc.VectorSubcoreMesh(core_axis_name="c", subcore_axis_name="s")

GPV = 264
TPW2 = (TPW + 1) // 2
TPWE = TPW2 * 2
PCHUNK = TPWE * NW
NL = C // 16
KA = K + K // 16


@functools.partial(
    pl.kernel,
    mesh=_mesh,
    compiler_params=pltpu.CompilerParams(needs_layout_passes=False),
    out_type=[
        jax.ShapeDtypeStruct((NW, G * C), jnp.float32),
        jax.ShapeDtypeStruct((NW, G * 16), jnp.float32),
    ],
    scratch_types=[
        pltpu.VMEM((K, C), jnp.float32),
        pltpu.VMEM((K, C), jnp.float32),
        pltpu.VMEM((KA, 16), jnp.int32),
        pltpu.VMEM((KA, 16), jnp.int32),
        pltpu.VMEM((GPV * C,), jnp.float32),
        pltpu.VMEM((GPV * 16,), jnp.float32),
        pltpu.SemaphoreType.DMA,
        pltpu.SemaphoreType.DMA,
        pltpu.SemaphoreType.DMA,
        pltpu.SemaphoreType.DMA,
    ],
)
def _segsum_sc(x_hbm, av_hbm, zc_hbm, z16_hbm,
               acc_out, cnt_out, xva, xvb, ava, avb, acc_f, cnt_f,
               sxa, sxb, saa, sab):
    cid = lax.axis_index("c")
    sid = lax.axis_index("s")
    wid = sid * NC + cid
    c0 = wid * TPWE

    pltpu.sync_copy(zc_hbm, acc_f)
    pltpu.sync_copy(z16_hbm, cnt_f)
    ones16 = jnp.ones((16,), jnp.float32)
    zero16 = jnp.zeros((16,), jnp.float32)
    prev0 = jnp.full((16,), TRASH * C, jnp.int32) + lax.iota(jnp.int32, 16)

    def start_loads(chunk, xbuf, abuf, sx, sa):
        chunk = jnp.minimum(chunk, PCHUNK - 1)
        start = jnp.minimum(chunk * K, N - K)
        pltpu.async_copy(x_hbm.at[pl.ds(start, K)], xbuf, sx)
        pltpu.async_copy(av_hbm.at[chunk], abuf, sa)

    def wait_loads(xbuf, abuf, sx, sa):
        pltpu.make_async_copy(x_hbm.at[pl.ds(0, K)], xbuf, sx).wait()
        pltpu.make_async_copy(av_hbm.at[0], abuf, sa).wait()

    def compute(xbuf, abuf, carry):
        for g in range(K // 16):
            plsc.addupdate_scatter(cnt_f, [abuf[K + g, :]], ones16)

        def slow_rows(g, carry):
            prev = carry[0]
            runs = list(carry[1:])
            for j in range(g * 16, g * 16 + 16):
                addr = abuf[j, :]
                ndiff = plsc.all_reduce_population_count(addr != prev)
                m = ndiff != 0
                for l in range(NL):
                    plsc.addupdate_scatter(acc_f, [prev + l * 16], runs[l],
                                           mask=m)
                for l in range(NL):
                    xr = xbuf[j, pl.ds(l * 16, 16)]
                    runs[l] = jnp.where(m, xr, runs[l] + xr)
                prev = addr
            return (prev, *runs)

        def fast_rows(g, carry):
            prev = carry[0]
            runs = list(carry[1:])
            a0 = abuf[g * 16, :]
            m = plsc.all_reduce_population_count(a0 != prev) != 0
            for l in range(NL):
                plsc.addupdate_scatter(acc_f, [prev + l * 16], runs[l], mask=m)
            for l in range(NL):
                s = xbuf[g * 16, pl.ds(l * 16, 16)]
                for j in range(1, 16):
                    s = s + xbuf[g * 16 + j, pl.ds(l * 16, 16)]
                runs[l] = jnp.where(m, s, runs[l] + s)
            return (a0, *runs)

        for g in range(K // 16):
            uni = jnp.sum(jnp.where(abuf[g * 16 + 15, :] != abuf[g * 16, :],
                                    1, 0), axis=0) == 0
            carry = lax.cond(uni,
                             lambda c, gg=g: fast_rows(gg, c),
                             lambda c, gg=g: slow_rows(gg, c),
                             carry)
        return carry

    start_loads(c0, xva, ava, sxa, saa)

    def pair_body(t2, carry):
        ca = c0 + 2 * t2
        start_loads(ca + 1, xvb, avb, sxb, sab)
        wait_loads(xva, ava, sxa, saa)
        carry = compute(xva, ava, carry)
        start_loads(ca + 2, xva, ava, sxa, saa)
        wait_loads(xvb, avb, sxb, sab)
        carry = compute(xvb, avb, carry)
        return carry

    final = lax.fori_loop(0, TPW2, pair_body, (prev0, *([zero16] * NL)))
    wait_loads(xva, ava, sxa, saa)
    fprev = final[0]
    for l in range(NL):
        plsc.addupdate_scatter(acc_f, [fprev + l * 16], final[1 + l])

    pltpu.sync_copy(acc_f.at[pl.ds(0, G * C)], acc_out.at[wid])
    pltpu.sync_copy(cnt_f.at[pl.ds(0, G * 16)], cnt_out.at[wid])


def _mlp_body(acc_ref, cnt_ref, w1_ref, a_ref, w2_ref, s_ref):
    seg = jnp.sum(acc_ref[...], axis=0)
    cnt_full = jnp.sum(cnt_ref[...], axis=0)
    cnt = jnp.maximum(jnp.sum(cnt_full, axis=1, keepdims=True), 1.0)
    x_avg = seg / cnt
    h = lax.dot_general(x_avg, w1_ref[...], (((1,), (1,)), ((), ())),
                        preferred_element_type=jnp.float32)
    a = a_ref[0]
    h = jnp.where(h >= 0, h, a * h)
    h = lax.dot_general(h, w2_ref[...], (((1,), (1,)), ((), ())),
                        preferred_element_type=jnp.float32)
    s_ref[...] = jax.nn.sigmoid(h)


def _scale_body(x_ref, b_ref, s_ref, o_ref):
    b = b_ref[0, 0, :]
    gi = lax.broadcasted_iota(jnp.int32, (BLK, G), 1)
    oh = jnp.where(gi == b[:, None], 1.0, 0.0).astype(jnp.float32)
    se = lax.dot_general(oh, s_ref[...], (((1,), (0,)), ((), ())),
                         preferred_element_type=jnp.float32)
    o_ref[...] = x_ref[...] * se


def kernel(x, batch, num_graphs, W1, a, W2):
    bi = jnp.minimum(batch, num_graphs - 1).astype(jnp.int32)

    nfull = N // K
    main = bi[: nfull * K].reshape(nfull, K)
    tail = bi[N - K:].reshape(1, K)
    tail_fresh = (jnp.arange(K) >= (nfull * K - (N - K)))[None, :]
    tail = jnp.where(tail_fresh, tail, TRASH)
    pad = jnp.full((PCHUNK - NCHUNK, K), TRASH, bi.dtype)
    bidx = jnp.concatenate([main, tail, pad], axis=0)[:PCHUNK].astype(jnp.int32)

    lane16 = jnp.arange(16, dtype=jnp.int32)
    av_tab = bidx[:, :, None] * C + lane16[None, None, :]
    ca_tab = (bidx * 16 + jnp.tile(lane16, K // 16)[None, :]).reshape(
        PCHUNK, K // 16, 16)
    aux = jnp.concatenate([av_tab, ca_tab], axis=1)

    zc = jnp.zeros((GPV * C,), jnp.float32)
    z16 = jnp.zeros((GPV * 16,), jnp.float32)

    acc, cnt = _segsum_sc(x, aux, zc, z16)
    acc = acc.reshape(NW, G, C)
    cnt = cnt.reshape(NW, G, 16)

    s = pl.pallas_call(
        _mlp_body,
        in_specs=[
            pl.BlockSpec((NW, G, C), lambda: (0, 0, 0)),
            pl.BlockSpec((NW, G, 16), lambda: (0, 0, 0)),
            pl.BlockSpec((H, C), lambda: (0, 0)),
            pl.BlockSpec(memory_space=pltpu.SMEM),
            pl.BlockSpec((C, H), lambda: (0, 0)),
        ],
        out_specs=pl.BlockSpec((G, C), lambda: (0, 0)),
        out_shape=jax.ShapeDtypeStruct((G, C), jnp.float32),
    )(acc, cnt, W1, a.reshape(1), W2)

    b3 = bi.reshape(NBLK, 1, BLK)
    out = pl.pallas_call(
        _scale_body,
        grid=(NBLK,),
        in_specs=[
            pl.BlockSpec((BLK, C), lambda i: (i, 0)),
            pl.BlockSpec((1, 1, BLK), lambda i: (i, 0, 0)),
            pl.BlockSpec((G, C), lambda i: (0, 0)),
        ],
        out_specs=pl.BlockSpec((BLK, C), lambda i: (i, 0)),
        out_shape=jax.ShapeDtypeStruct((N, C), jnp.float32),
    )(x, b3, s)
    return out


# speedup vs baseline: 2.1892x; 1.5091x over previous
import functools

import jax
import jax.numpy as jnp
from jax import lax
from jax.experimental import pallas as pl
from jax.experimental.pallas import tpu as pltpu
from jax.experimental.pallas import tpu_sc as plsc

N = 100000
C = 256
G = 256
H = 16

NC = 2
NS = 16
NW = NC * NS

NSC = 20000
NTC = N - NSC

K = 32
NCHUNK = NSC // K
TRASH = G
GPV = 264
TPW = (NCHUNK + NW - 1) // NW
TPW2 = (TPW + 1) // 2
TPWE = TPW2 * 2
PCHUNK = TPWE * NW
NL = C // 16
KA = K + K // 16

BLK = 2000
NBLK_A = NTC // BLK
NBLK = N // BLK

_mesh = plsc.VectorSubcoreMesh(core_axis_name="c", subcore_axis_name="s")


@functools.partial(
    pl.kernel,
    mesh=_mesh,
    compiler_params=pltpu.CompilerParams(needs_layout_passes=False),
    out_type=[
        jax.ShapeDtypeStruct((NW, G * C), jnp.float32),
        jax.ShapeDtypeStruct((NW, G * 16), jnp.float32),
    ],
    scratch_types=[
        pltpu.VMEM((K, C), jnp.float32),
        pltpu.VMEM((K, C), jnp.float32),
        pltpu.VMEM((KA, 16), jnp.int32),
        pltpu.VMEM((KA, 16), jnp.int32),
        pltpu.VMEM((GPV * C,), jnp.float32),
        pltpu.VMEM((GPV * 16,), jnp.float32),
        pltpu.SemaphoreType.DMA,
        pltpu.SemaphoreType.DMA,
        pltpu.SemaphoreType.DMA,
        pltpu.SemaphoreType.DMA,
    ],
)
def _segsum_sc(x_hbm, av_hbm, zc_hbm, z16_hbm,
               acc_out, cnt_out, xva, xvb, ava, avb, acc_f, cnt_f,
               sxa, sxb, saa, sab):
    cid = lax.axis_index("c")
    sid = lax.axis_index("s")
    wid = sid * NC + cid
    c0 = wid * TPWE

    pltpu.sync_copy(zc_hbm, acc_f)
    pltpu.sync_copy(z16_hbm, cnt_f)
    ones16 = jnp.ones((16,), jnp.float32)
    zero16 = jnp.zeros((16,), jnp.float32)
    prev0 = jnp.full((16,), TRASH * C, jnp.int32) + lax.iota(jnp.int32, 16)

    def start_loads(chunk, xbuf, abuf, sx, sa):
        chunk = jnp.minimum(chunk, PCHUNK - 1)
        start = jnp.minimum(chunk * K, NSC - K)
        pltpu.async_copy(x_hbm.at[pl.ds(start, K)], xbuf, sx)
        pltpu.async_copy(av_hbm.at[chunk], abuf, sa)

    def wait_loads(xbuf, abuf, sx, sa):
        pltpu.make_async_copy(x_hbm.at[pl.ds(0, K)], xbuf, sx).wait()
        pltpu.make_async_copy(av_hbm.at[0], abuf, sa).wait()

    def compute(xbuf, abuf, carry):
        for g in range(K // 16):
            plsc.addupdate_scatter(cnt_f, [abuf[K + g, :]], ones16)

        def slow_rows(g, carry):
            prev = carry[0]
            runs = list(carry[1:])
            for j in range(g * 16, g * 16 + 16):
                addr = abuf[j, :]
                m = plsc.all_reduce_population_count(addr != prev) != 0
                for l in range(NL):
                    plsc.addupdate_scatter(acc_f, [prev + l * 16], runs[l],
                                           mask=m)
                for l in range(NL):
                    xr = xbuf[j, pl.ds(l * 16, 16)]
                    runs[l] = jnp.where(m, xr, runs[l] + xr)
                prev = addr
            return (prev, *runs)

        def fast_rows(g, carry):
            prev = carry[0]
            runs = list(carry[1:])
            a0 = abuf[g * 16, :]
            m = plsc.all_reduce_population_count(a0 != prev) != 0
            for l in range(NL):
                plsc.addupdate_scatter(acc_f, [prev + l * 16], runs[l], mask=m)
            for l in range(NL):
                s = xbuf[g * 16, pl.ds(l * 16, 16)]
                for j in range(1, 16):
                    s = s + xbuf[g * 16 + j, pl.ds(l * 16, 16)]
                runs[l] = jnp.where(m, s, runs[l] + s)
            return (a0, *runs)

        for g in range(K // 16):
            uni = jnp.sum(jnp.where(abuf[g * 16 + 15, :] != abuf[g * 16, :],
                                    1, 0), axis=0) == 0
            carry = lax.cond(uni,
                             lambda c, gg=g: fast_rows(gg, c),
                             lambda c, gg=g: slow_rows(gg, c),
                             carry)
        return carry

    start_loads(c0, xva, ava, sxa, saa)

    def pair_body(t2, carry):
        ca = c0 + 2 * t2
        start_loads(ca + 1, xvb, avb, sxb, sab)
        wait_loads(xva, ava, sxa, saa)
        carry = compute(xva, ava, carry)
        start_loads(ca + 2, xva, ava, sxa, saa)
        wait_loads(xvb, avb, sxb, sab)
        carry = compute(xvb, avb, carry)
        return carry

    final = lax.fori_loop(0, TPW2, pair_body, (prev0, *([zero16] * NL)))
    wait_loads(xva, ava, sxa, saa)
    fprev = final[0]
    for l in range(NL):
        plsc.addupdate_scatter(acc_f, [fprev + l * 16], final[1 + l])

    pltpu.sync_copy(acc_f.at[pl.ds(0, G * C)], acc_out.at[wid])
    pltpu.sync_copy(cnt_f.at[pl.ds(0, G * 16)], cnt_out.at[wid])


def _parta_body(x_ref, b_ref, seg_ref, cnt_ref, seg_acc, cnt_acc):
    i = pl.program_id(0)

    @pl.when(i == 0)
    def _init():
        seg_acc[...] = jnp.zeros_like(seg_acc)
        cnt_acc[...] = jnp.zeros_like(cnt_acc)

    b = b_ref[0, 0, :]
    gi = lax.broadcasted_iota(jnp.int32, (BLK, G), 1)
    oh = jnp.where(gi == b[:, None], 1.0, 0.0).astype(jnp.bfloat16)
    seg_acc[...] += lax.dot_general(
        oh, x_ref[...].astype(jnp.bfloat16), (((0,), (0,)), ((), ())),
        preferred_element_type=jnp.float32)
    cnt_acc[...] += jnp.sum(oh.astype(jnp.float32), axis=0, keepdims=True)

    @pl.when(i == NBLK_A - 1)
    def _finish():
        seg_ref[...] = seg_acc[...]
        cnt_ref[...] = cnt_acc[...]


def _mlp_body(acc_ref, cnt_ref, sega_ref, cnta_ref, w1_ref, a_ref, w2_ref,
              s_ref):
    seg = jnp.sum(acc_ref[...], axis=0) + sega_ref[...]
    cnt_l = jnp.sum(cnt_ref[...], axis=0)
    cnt = jnp.sum(cnt_l, axis=1, keepdims=True) + cnta_ref[...].reshape(G, 1)
    cnt = jnp.maximum(cnt, 1.0)
    x_avg = seg / cnt
    h = lax.dot_general(x_avg, w1_ref[...], (((1,), (1,)), ((), ())),
                        preferred_element_type=jnp.float32)
    a = a_ref[0]
    h = jnp.where(h >= 0, h, a * h)
    h = lax.dot_general(h, w2_ref[...], (((1,), (1,)), ((), ())),
                        preferred_element_type=jnp.float32)
    s_ref[...] = jax.nn.sigmoid(h)


def _scale_body(x_ref, b_ref, s_ref, o_ref):
    b = b_ref[0, 0, :]
    gi = lax.broadcasted_iota(jnp.int32, (BLK, G), 1)
    oh = jnp.where(gi == b[:, None], 1.0, 0.0).astype(jnp.bfloat16)
    se = lax.dot_general(oh, s_ref[...].astype(jnp.bfloat16),
                         (((1,), (0,)), ((), ())),
                         preferred_element_type=jnp.float32)
    o_ref[...] = x_ref[...] * se


def kernel(x, batch, num_graphs, W1, a, W2):
    bi = jnp.minimum(batch, num_graphs - 1).astype(jnp.int32)

    main = bi[:NSC].reshape(NCHUNK, K)
    pad = jnp.full((PCHUNK - NCHUNK, K), TRASH, jnp.int32)
    bidx = jnp.concatenate([main, pad], axis=0)

    lane16 = jnp.arange(16, dtype=jnp.int32)
    av_tab = bidx[:, :, None] * C + lane16[None, None, :]
    ca_tab = (bidx * 16 + jnp.tile(lane16, K // 16)[None, :]).reshape(
        PCHUNK, K // 16, 16)
    aux = jnp.concatenate([av_tab, ca_tab], axis=1)

    zc = jnp.zeros((GPV * C,), jnp.float32)
    z16 = jnp.zeros((GPV * 16,), jnp.float32)

    acc, cnt = _segsum_sc(x, aux, zc, z16)
    acc = acc.reshape(NW, G, C)
    cnt = cnt.reshape(NW, G, 16)

    b3a = bi[NSC:].reshape(NBLK_A, 1, BLK)
    sega, cnta = pl.pallas_call(
        _parta_body,
        grid=(NBLK_A,),
        in_specs=[
            pl.BlockSpec((BLK, C), lambda i: (i + NSC // BLK, 0)),
            pl.BlockSpec((1, 1, BLK), lambda i: (i, 0, 0)),
        ],
        out_specs=[
            pl.BlockSpec((G, C), lambda i: (0, 0)),
            pl.BlockSpec((1, G), lambda i: (0, 0)),
        ],
        out_shape=[
            jax.ShapeDtypeStruct((G, C), jnp.float32),
            jax.ShapeDtypeStruct((1, G), jnp.float32),
        ],
        scratch_shapes=[
            pltpu.VMEM((G, C), jnp.float32),
            pltpu.VMEM((1, G), jnp.float32),
        ],
    )(x, b3a)

    s = pl.pallas_call(
        _mlp_body,
        in_specs=[
            pl.BlockSpec((NW, G, C), lambda: (0, 0, 0)),
            pl.BlockSpec((NW, G, 16), lambda: (0, 0, 0)),
            pl.BlockSpec((G, C), lambda: (0, 0)),
            pl.BlockSpec((1, G), lambda: (0, 0)),
            pl.BlockSpec((H, C), lambda: (0, 0)),
            pl.BlockSpec(memory_space=pltpu.SMEM),
            pl.BlockSpec((C, H), lambda: (0, 0)),
        ],
        out_specs=pl.BlockSpec((G, C), lambda: (0, 0)),
        out_shape=jax.ShapeDtypeStruct((G, C), jnp.float32),
    )(acc, cnt, sega, cnta, W1, a.reshape(1), W2)

    b3 = bi.reshape(NBLK, 1, BLK)
    out = pl.pallas_call(
        _scale_body,
        grid=(NBLK,),
        in_specs=[
            pl.BlockSpec((BLK, C), lambda i: (i, 0)),
            pl.BlockSpec((1, 1, BLK), lambda i: (i, 0, 0)),
            pl.BlockSpec((G, C), lambda i: (0, 0)),
        ],
        out_specs=pl.BlockSpec((BLK, C), lambda i: (i, 0)),
        out_shape=jax.ShapeDtypeStruct((N, C), jnp.float32),
    )(x, b3, s)
    return out


# speedup vs baseline: 2.3171x; 1.0584x over previous
import functools

import jax
import jax.numpy as jnp
from jax import lax
from jax.experimental import pallas as pl
from jax.experimental.pallas import tpu as pltpu
from jax.experimental.pallas import tpu_sc as plsc

N = 100000
C = 256
G = 256
H = 16

NC = 2
NS = 16
NW = NC * NS

NSC = 12000
NTC = N - NSC

K = 32
NCHUNK = NSC // K
TRASH = G
GPV = 264
TPW = (NCHUNK + NW - 1) // NW
TPW2 = (TPW + 1) // 2
TPWE = TPW2 * 2
PCHUNK = TPWE * NW
NL = C // 16
KA = K + K // 16

BLK = 2000
NBLK_A = NTC // BLK
NBLK = N // BLK

_mesh = plsc.VectorSubcoreMesh(core_axis_name="c", subcore_axis_name="s")


@functools.partial(
    pl.kernel,
    mesh=_mesh,
    compiler_params=pltpu.CompilerParams(needs_layout_passes=False),
    out_type=[
        jax.ShapeDtypeStruct((NW, G * C), jnp.float32),
        jax.ShapeDtypeStruct((NW, G * 16), jnp.float32),
    ],
    scratch_types=[
        pltpu.VMEM((K, C), jnp.float32),
        pltpu.VMEM((K, C), jnp.float32),
        pltpu.VMEM((KA, 16), jnp.int32),
        pltpu.VMEM((KA, 16), jnp.int32),
        pltpu.VMEM((GPV * C,), jnp.float32),
        pltpu.VMEM((GPV * 16,), jnp.float32),
        pltpu.SemaphoreType.DMA,
        pltpu.SemaphoreType.DMA,
        pltpu.SemaphoreType.DMA,
        pltpu.SemaphoreType.DMA,
    ],
)
def _segsum_sc(x_hbm, av_hbm, zc_hbm, z16_hbm,
               acc_out, cnt_out, xva, xvb, ava, avb, acc_f, cnt_f,
               sxa, sxb, saa, sab):
    cid = lax.axis_index("c")
    sid = lax.axis_index("s")
    wid = sid * NC + cid
    c0 = wid * TPWE

    pltpu.sync_copy(zc_hbm, acc_f)
    pltpu.sync_copy(z16_hbm, cnt_f)
    ones16 = jnp.ones((16,), jnp.float32)
    zero16 = jnp.zeros((16,), jnp.float32)
    prev0 = jnp.full((16,), TRASH * C, jnp.int32) + lax.iota(jnp.int32, 16)

    def start_loads(chunk, xbuf, abuf, sx, sa):
        chunk = jnp.minimum(chunk, PCHUNK - 1)
        start = jnp.minimum(chunk * K, NSC - K)
        pltpu.async_copy(x_hbm.at[pl.ds(start, K)], xbuf, sx)
        pltpu.async_copy(av_hbm.at[chunk], abuf, sa)

    def wait_loads(xbuf, abuf, sx, sa):
        pltpu.make_async_copy(x_hbm.at[pl.ds(0, K)], xbuf, sx).wait()
        pltpu.make_async_copy(av_hbm.at[0], abuf, sa).wait()

    def compute(xbuf, abuf, carry):
        for g in range(K // 16):
            plsc.addupdate_scatter(cnt_f, [abuf[K + g, :]], ones16)

        def slow_rows(g, carry):
            prev = carry[0]
            runs = list(carry[1:])
            for j in range(g * 16, g * 16 + 16):
                addr = abuf[j, :]
                m = plsc.all_reduce_population_count(addr != prev) != 0
                for l in range(NL):
                    plsc.addupdate_scatter(acc_f, [prev + l * 16], runs[l],
                                           mask=m)
                for l in range(NL):
                    xr = xbuf[j, pl.ds(l * 16, 16)]
                    runs[l] = jnp.where(m, xr, runs[l] + xr)
                prev = addr
            return (prev, *runs)

        def fast_rows(g, carry):
            prev = carry[0]
            runs = list(carry[1:])
            a0 = abuf[g * 16, :]
            m = plsc.all_reduce_population_count(a0 != prev) != 0
            for l in range(NL):
                plsc.addupdate_scatter(acc_f, [prev + l * 16], runs[l], mask=m)
            for l in range(NL):
                s = xbuf[g * 16, pl.ds(l * 16, 16)]
                for j in range(1, 16):
                    s = s + xbuf[g * 16 + j, pl.ds(l * 16, 16)]
                runs[l] = jnp.where(m, s, runs[l] + s)
            return (a0, *runs)

        for g in range(K // 16):
            uni = jnp.sum(jnp.where(abuf[g * 16 + 15, :] != abuf[g * 16, :],
                                    1, 0), axis=0) == 0
            carry = lax.cond(uni,
                             lambda c, gg=g: fast_rows(gg, c),
                             lambda c, gg=g: slow_rows(gg, c),
                             carry)
        return carry

    start_loads(c0, xva, ava, sxa, saa)

    def pair_body(t2, carry):
        ca = c0 + 2 * t2
        start_loads(ca + 1, xvb, avb, sxb, sab)
        wait_loads(xva, ava, sxa, saa)
        carry = compute(xva, ava, carry)
        start_loads(ca + 2, xva, ava, sxa, saa)
        wait_loads(xvb, avb, sxb, sab)
        carry = compute(xvb, avb, carry)
        return carry

    final = lax.fori_loop(0, TPW2, pair_body, (prev0, *([zero16] * NL)))
    wait_loads(xva, ava, sxa, saa)
    fprev = final[0]
    for l in range(NL):
        plsc.addupdate_scatter(acc_f, [fprev + l * 16], final[1 + l])

    pltpu.sync_copy(acc_f.at[pl.ds(0, G * C)], acc_out.at[wid])
    pltpu.sync_copy(cnt_f.at[pl.ds(0, G * 16)], cnt_out.at[wid])


def _parta_body(x_ref, b_ref, seg_ref, cnt_ref, seg_acc, cnt_acc):
    i = pl.program_id(0)

    @pl.when(i == 0)
    def _init():
        seg_acc[...] = jnp.zeros_like(seg_acc)
        cnt_acc[...] = jnp.zeros_like(cnt_acc)

    b = b_ref[0, 0, :]
    gi = lax.broadcasted_iota(jnp.int32, (BLK, G), 1)
    oh = jnp.where(gi == b[:, None], 1.0, 0.0).astype(jnp.bfloat16)
    seg_acc[...] += lax.dot_general(
        oh, x_ref[...].astype(jnp.bfloat16), (((0,), (0,)), ((), ())),
        preferred_element_type=jnp.float32)
    cnt_acc[...] += jnp.sum(oh.astype(jnp.float32), axis=0, keepdims=True)

    @pl.when(i == NBLK_A - 1)
    def _finish():
        seg_ref[...] = seg_acc[...]
        cnt_ref[...] = cnt_acc[...]


def _mlp_body(acc_ref, cnt_ref, sega_ref, cnta_ref, w1_ref, a_ref, w2_ref,
              s_ref):
    seg = jnp.sum(acc_ref[...], axis=0) + sega_ref[...]
    cnt_l = jnp.sum(cnt_ref[...], axis=0)
    cnt = jnp.sum(cnt_l, axis=1, keepdims=True) + cnta_ref[...].reshape(G, 1)
    cnt = jnp.maximum(cnt, 1.0)
    x_avg = seg / cnt
    h = lax.dot_general(x_avg, w1_ref[...], (((1,), (1,)), ((), ())),
                        preferred_element_type=jnp.float32)
    a = a_ref[0]
    h = jnp.where(h >= 0, h, a * h)
    h = lax.dot_general(h, w2_ref[...], (((1,), (1,)), ((), ())),
                        preferred_element_type=jnp.float32)
    s_ref[...] = jax.nn.sigmoid(h)


def _scale_body(x_ref, b_ref, s_ref, o_ref):
    b = b_ref[0, 0, :]
    gi = lax.broadcasted_iota(jnp.int32, (BLK, G), 1)
    oh = jnp.where(gi == b[:, None], 1.0, 0.0).astype(jnp.bfloat16)
    se = lax.dot_general(oh, s_ref[...].astype(jnp.bfloat16),
                         (((1,), (0,)), ((), ())),
                         preferred_element_type=jnp.float32)
    o_ref[...] = x_ref[...] * se


def kernel(x, batch, num_graphs, W1, a, W2):
    bi = jnp.minimum(batch, num_graphs - 1).astype(jnp.int32)

    main = bi[:NSC].reshape(NCHUNK, K)
    pad = jnp.full((PCHUNK - NCHUNK, K), TRASH, jnp.int32)
    bidx = jnp.concatenate([main, pad], axis=0)

    lane16 = jnp.arange(16, dtype=jnp.int32)
    av_tab = bidx[:, :, None] * C + lane16[None, None, :]
    ca_tab = (bidx * 16 + jnp.tile(lane16, K // 16)[None, :]).reshape(
        PCHUNK, K // 16, 16)
    aux = jnp.concatenate([av_tab, ca_tab], axis=1)

    zc = jnp.zeros((GPV * C,), jnp.float32)
    z16 = jnp.zeros((GPV * 16,), jnp.float32)

    acc, cnt = _segsum_sc(x, aux, zc, z16)
    acc = acc.reshape(NW, G, C)
    cnt = cnt.reshape(NW, G, 16)

    b3a = bi[NSC:].reshape(NBLK_A, 1, BLK)
    sega, cnta = pl.pallas_call(
        _parta_body,
        grid=(NBLK_A,),
        in_specs=[
            pl.BlockSpec((BLK, C), lambda i: (i + NSC // BLK, 0)),
            pl.BlockSpec((1, 1, BLK), lambda i: (i, 0, 0)),
        ],
        out_specs=[
            pl.BlockSpec((G, C), lambda i: (0, 0)),
            pl.BlockSpec((1, G), lambda i: (0, 0)),
        ],
        out_shape=[
            jax.ShapeDtypeStruct((G, C), jnp.float32),
            jax.ShapeDtypeStruct((1, G), jnp.float32),
        ],
        scratch_shapes=[
            pltpu.VMEM((G, C), jnp.float32),
            pltpu.VMEM((1, G), jnp.float32),
        ],
    )(x, b3a)

    s = pl.pallas_call(
        _mlp_body,
        in_specs=[
            pl.BlockSpec((NW, G, C), lambda: (0, 0, 0)),
            pl.BlockSpec((NW, G, 16), lambda: (0, 0, 0)),
            pl.BlockSpec((G, C), lambda: (0, 0)),
            pl.BlockSpec((1, G), lambda: (0, 0)),
            pl.BlockSpec((H, C), lambda: (0, 0)),
            pl.BlockSpec(memory_space=pltpu.SMEM),
            pl.BlockSpec((C, H), lambda: (0, 0)),
        ],
        out_specs=pl.BlockSpec((G, C), lambda: (0, 0)),
        out_shape=jax.ShapeDtypeStruct((G, C), jnp.float32),
    )(acc, cnt, sega, cnta, W1, a.reshape(1), W2)

    b3 = bi.reshape(NBLK, 1, BLK)
    out = pl.pallas_call(
        _scale_body,
        grid=(NBLK,),
        in_specs=[
            pl.BlockSpec((BLK, C), lambda i: (i, 0)),
            pl.BlockSpec((1, 1, BLK), lambda i: (i, 0, 0)),
            pl.BlockSpec((G, C), lambda i: (0, 0)),
        ],
        out_specs=pl.BlockSpec((BLK, C), lambda i: (i, 0)),
        out_shape=jax.ShapeDtypeStruct((N, C), jnp.float32),
    )(x, b3, s)
    return out


# speedup vs baseline: 2.3874x; 1.0304x over previous
import functools

import jax
import jax.numpy as jnp
from jax import lax
from jax.experimental import pallas as pl
from jax.experimental.pallas import tpu as pltpu
from jax.experimental.pallas import tpu_sc as plsc

N = 100000
C = 256
G = 256
H = 16

NC = 2
NS = 16
NW = NC * NS

NSC = 8000
NTC = N - NSC

K = 32
NCHUNK = NSC // K
TRASH = G
GPV = 264
TPW = (NCHUNK + NW - 1) // NW
TPW2 = (TPW + 1) // 2
TPWE = TPW2 * 2
PCHUNK = TPWE * NW
NL = C // 16
KA = K + K // 16

BLK = 2000
NBLK_A = NTC // BLK
NBLK = N // BLK

_mesh = plsc.VectorSubcoreMesh(core_axis_name="c", subcore_axis_name="s")


@functools.partial(
    pl.kernel,
    mesh=_mesh,
    compiler_params=pltpu.CompilerParams(needs_layout_passes=False),
    out_type=[
        jax.ShapeDtypeStruct((NW, G * C), jnp.float32),
        jax.ShapeDtypeStruct((NW, G * 16), jnp.float32),
    ],
    scratch_types=[
        pltpu.VMEM((K, C), jnp.float32),
        pltpu.VMEM((K, C), jnp.float32),
        pltpu.VMEM((KA, 16), jnp.int32),
        pltpu.VMEM((KA, 16), jnp.int32),
        pltpu.VMEM((GPV * C,), jnp.float32),
        pltpu.VMEM((GPV * 16,), jnp.float32),
        pltpu.SemaphoreType.DMA,
        pltpu.SemaphoreType.DMA,
        pltpu.SemaphoreType.DMA,
        pltpu.SemaphoreType.DMA,
    ],
)
def _segsum_sc(x_hbm, av_hbm, zc_hbm, z16_hbm,
               acc_out, cnt_out, xva, xvb, ava, avb, acc_f, cnt_f,
               sxa, sxb, saa, sab):
    cid = lax.axis_index("c")
    sid = lax.axis_index("s")
    wid = sid * NC + cid
    c0 = wid * TPWE

    pltpu.sync_copy(zc_hbm, acc_f)
    pltpu.sync_copy(z16_hbm, cnt_f)
    ones16 = jnp.ones((16,), jnp.float32)
    zero16 = jnp.zeros((16,), jnp.float32)
    prev0 = jnp.full((16,), TRASH * C, jnp.int32) + lax.iota(jnp.int32, 16)

    def start_loads(chunk, xbuf, abuf, sx, sa):
        chunk = jnp.minimum(chunk, PCHUNK - 1)
        start = jnp.minimum(chunk * K, NSC - K)
        pltpu.async_copy(x_hbm.at[pl.ds(start, K)], xbuf, sx)
        pltpu.async_copy(av_hbm.at[chunk], abuf, sa)

    def wait_loads(xbuf, abuf, sx, sa):
        pltpu.make_async_copy(x_hbm.at[pl.ds(0, K)], xbuf, sx).wait()
        pltpu.make_async_copy(av_hbm.at[0], abuf, sa).wait()

    def compute(xbuf, abuf, carry):
        for g in range(K // 16):
            plsc.addupdate_scatter(cnt_f, [abuf[K + g, :]], ones16)

        def slow_rows(g, carry):
            prev = carry[0]
            runs = list(carry[1:])
            for j in range(g * 16, g * 16 + 16):
                addr = abuf[j, :]
                m = plsc.all_reduce_population_count(addr != prev) != 0
                for l in range(NL):
                    plsc.addupdate_scatter(acc_f, [prev + l * 16], runs[l],
                                           mask=m)
                for l in range(NL):
                    xr = xbuf[j, pl.ds(l * 16, 16)]
                    runs[l] = jnp.where(m, xr, runs[l] + xr)
                prev = addr
            return (prev, *runs)

        def fast_rows(g, carry):
            prev = carry[0]
            runs = list(carry[1:])
            a0 = abuf[g * 16, :]
            m = plsc.all_reduce_population_count(a0 != prev) != 0
            for l in range(NL):
                plsc.addupdate_scatter(acc_f, [prev + l * 16], runs[l], mask=m)
            for l in range(NL):
                s = xbuf[g * 16, pl.ds(l * 16, 16)]
                for j in range(1, 16):
                    s = s + xbuf[g * 16 + j, pl.ds(l * 16, 16)]
                runs[l] = jnp.where(m, s, runs[l] + s)
            return (a0, *runs)

        for g in range(K // 16):
            uni = jnp.sum(jnp.where(abuf[g * 16 + 15, :] != abuf[g * 16, :],
                                    1, 0), axis=0) == 0
            carry = lax.cond(uni,
                             lambda c, gg=g: fast_rows(gg, c),
                             lambda c, gg=g: slow_rows(gg, c),
                             carry)
        return carry

    start_loads(c0, xva, ava, sxa, saa)

    def pair_body(t2, carry):
        ca = c0 + 2 * t2
        start_loads(ca + 1, xvb, avb, sxb, sab)
        wait_loads(xva, ava, sxa, saa)
        carry = compute(xva, ava, carry)
        start_loads(ca + 2, xva, ava, sxa, saa)
        wait_loads(xvb, avb, sxb, sab)
        carry = compute(xvb, avb, carry)
        return carry

    final = lax.fori_loop(0, TPW2, pair_body, (prev0, *([zero16] * NL)))
    wait_loads(xva, ava, sxa, saa)
    fprev = final[0]
    for l in range(NL):
        plsc.addupdate_scatter(acc_f, [fprev + l * 16], final[1 + l])

    pltpu.sync_copy(acc_f.at[pl.ds(0, G * C)], acc_out.at[wid])
    pltpu.sync_copy(cnt_f.at[pl.ds(0, G * 16)], cnt_out.at[wid])


def _parta_body(x_ref, b_ref, seg_ref, cnt_ref, seg_acc, cnt_acc):
    i = pl.program_id(0)

    @pl.when(i == 0)
    def _init():
        seg_acc[...] = jnp.zeros_like(seg_acc)
        cnt_acc[...] = jnp.zeros_like(cnt_acc)

    b = b_ref[0, 0, :]
    gi = lax.broadcasted_iota(jnp.int32, (BLK, G), 1)
    oh = jnp.where(gi == b[:, None], 1.0, 0.0).astype(jnp.bfloat16)
    seg_acc[...] += lax.dot_general(
        oh, x_ref[...].astype(jnp.bfloat16), (((0,), (0,)), ((), ())),
        preferred_element_type=jnp.float32)
    cnt_acc[...] += jnp.sum(oh.astype(jnp.float32), axis=0, keepdims=True)

    @pl.when(i == NBLK_A - 1)
    def _finish():
        seg_ref[...] = seg_acc[...]
        cnt_ref[...] = cnt_acc[...]


def _mlp_body(acc_ref, cnt_ref, sega_ref, cnta_ref, w1_ref, a_ref, w2_ref,
              s_ref):
    seg = jnp.sum(acc_ref[...], axis=0) + sega_ref[...]
    cnt_l = jnp.sum(cnt_ref[...], axis=0)
    cnt = jnp.sum(cnt_l, axis=1, keepdims=True) + cnta_ref[...].reshape(G, 1)
    cnt = jnp.maximum(cnt, 1.0)
    x_avg = seg / cnt
    h = lax.dot_general(x_avg, w1_ref[...], (((1,), (1,)), ((), ())),
                        preferred_element_type=jnp.float32)
    a = a_ref[0]
    h = jnp.where(h >= 0, h, a * h)
    h = lax.dot_general(h, w2_ref[...], (((1,), (1,)), ((), ())),
                        preferred_element_type=jnp.float32)
    s_ref[...] = jax.nn.sigmoid(h)


def _scale_body(x_ref, b_ref, s_ref, o_ref):
    b = b_ref[0, 0, :]
    gi = lax.broadcasted_iota(jnp.int32, (BLK, G), 1)
    oh = jnp.where(gi == b[:, None], 1.0, 0.0).astype(jnp.bfloat16)
    se = lax.dot_general(oh, s_ref[...].astype(jnp.bfloat16),
                         (((1,), (0,)), ((), ())),
                         preferred_element_type=jnp.float32)
    o_ref[...] = x_ref[...] * se


def kernel(x, batch, num_graphs, W1, a, W2):
    bi = jnp.minimum(batch, num_graphs - 1).astype(jnp.int32)

    main = bi[:NSC].reshape(NCHUNK, K)
    pad = jnp.full((PCHUNK - NCHUNK, K), TRASH, jnp.int32)
    bidx = jnp.concatenate([main, pad], axis=0)

    lane16 = jnp.arange(16, dtype=jnp.int32)
    av_tab = bidx[:, :, None] * C + lane16[None, None, :]
    ca_tab = (bidx * 16 + jnp.tile(lane16, K // 16)[None, :]).reshape(
        PCHUNK, K // 16, 16)
    aux = jnp.concatenate([av_tab, ca_tab], axis=1)

    zc = jnp.zeros((GPV * C,), jnp.float32)
    z16 = jnp.zeros((GPV * 16,), jnp.float32)

    acc, cnt = _segsum_sc(x, aux, zc, z16)
    acc = acc.reshape(NW, G, C)
    cnt = cnt.reshape(NW, G, 16)

    b3a = bi[NSC:].reshape(NBLK_A, 1, BLK)
    sega, cnta = pl.pallas_call(
        _parta_body,
        grid=(NBLK_A,),
        in_specs=[
            pl.BlockSpec((BLK, C), lambda i: (i + NSC // BLK, 0)),
            pl.BlockSpec((1, 1, BLK), lambda i: (i, 0, 0)),
        ],
        out_specs=[
            pl.BlockSpec((G, C), lambda i: (0, 0)),
            pl.BlockSpec((1, G), lambda i: (0, 0)),
        ],
        out_shape=[
            jax.ShapeDtypeStruct((G, C), jnp.float32),
            jax.ShapeDtypeStruct((1, G), jnp.float32),
        ],
        scratch_shapes=[
            pltpu.VMEM((G, C), jnp.float32),
            pltpu.VMEM((1, G), jnp.float32),
        ],
    )(x, b3a)

    s = pl.pallas_call(
        _mlp_body,
        in_specs=[
            pl.BlockSpec((NW, G, C), lambda: (0, 0, 0)),
            pl.BlockSpec((NW, G, 16), lambda: (0, 0, 0)),
            pl.BlockSpec((G, C), lambda: (0, 0)),
            pl.BlockSpec((1, G), lambda: (0, 0)),
            pl.BlockSpec((H, C), lambda: (0, 0)),
            pl.BlockSpec(memory_space=pltpu.SMEM),
            pl.BlockSpec((C, H), lambda: (0, 0)),
        ],
        out_specs=pl.BlockSpec((G, C), lambda: (0, 0)),
        out_shape=jax.ShapeDtypeStruct((G, C), jnp.float32),
    )(acc, cnt, sega, cnta, W1, a.reshape(1), W2)

    b3 = bi.reshape(NBLK, 1, BLK)
    out = pl.pallas_call(
        _scale_body,
        grid=(NBLK,),
        in_specs=[
            pl.BlockSpec((BLK, C), lambda i: (i, 0)),
            pl.BlockSpec((1, 1, BLK), lambda i: (i, 0, 0)),
            pl.BlockSpec((G, C), lambda i: (0, 0)),
        ],
        out_specs=pl.BlockSpec((BLK, C), lambda i: (i, 0)),
        out_shape=jax.ShapeDtypeStruct((N, C), jnp.float32),
    )(x, b3, s)
    return out


# speedup vs baseline: 2.7080x; 1.1343x over previous
import functools

import jax
import jax.numpy as jnp
from jax import lax
from jax.experimental import pallas as pl
from jax.experimental.pallas import tpu as pltpu
from jax.experimental.pallas import tpu_sc as plsc

N = 100000
C = 256
G = 256
H = 16

NC = 2
NS = 16
NW = NC * NS

NSC = 8000
NTC = N - NSC

K = 32
NCHUNK = NSC // K
TRASH = G
GPV = 264
TPW = (NCHUNK + NW - 1) // NW
TPW2 = (TPW + 1) // 2
TPWE = TPW2 * 2
PCHUNK = TPWE * NW
NL = C // 16
KA = K + K // 16

BLK = 4000
NBLK_A = NTC // BLK
NBLK = N // BLK

_mesh = plsc.VectorSubcoreMesh(core_axis_name="c", subcore_axis_name="s")


@functools.partial(
    pl.kernel,
    mesh=_mesh,
    compiler_params=pltpu.CompilerParams(needs_layout_passes=False),
    out_type=[
        jax.ShapeDtypeStruct((NW, G * C), jnp.float32),
        jax.ShapeDtypeStruct((NW, G * 16), jnp.float32),
    ],
    scratch_types=[
        pltpu.VMEM((K, C), jnp.float32),
        pltpu.VMEM((K, C), jnp.float32),
        pltpu.VMEM((KA, 16), jnp.int32),
        pltpu.VMEM((KA, 16), jnp.int32),
        pltpu.VMEM((GPV * C,), jnp.float32),
        pltpu.VMEM((GPV * 16,), jnp.float32),
        pltpu.SemaphoreType.DMA,
        pltpu.SemaphoreType.DMA,
        pltpu.SemaphoreType.DMA,
        pltpu.SemaphoreType.DMA,
    ],
)
def _segsum_sc(x_hbm, av_hbm, zc_hbm, z16_hbm,
               acc_out, cnt_out, xva, xvb, ava, avb, acc_f, cnt_f,
               sxa, sxb, saa, sab):
    cid = lax.axis_index("c")
    sid = lax.axis_index("s")
    wid = sid * NC + cid
    c0 = wid * TPWE

    pltpu.sync_copy(zc_hbm, acc_f)
    pltpu.sync_copy(z16_hbm, cnt_f)
    ones16 = jnp.ones((16,), jnp.float32)
    zero16 = jnp.zeros((16,), jnp.float32)
    prev0 = jnp.full((16,), TRASH * C, jnp.int32) + lax.iota(jnp.int32, 16)

    def start_loads(chunk, xbuf, abuf, sx, sa):
        chunk = jnp.minimum(chunk, PCHUNK - 1)
        start = jnp.minimum(chunk * K, NSC - K)
        pltpu.async_copy(x_hbm.at[pl.ds(start, K)], xbuf, sx)
        pltpu.async_copy(av_hbm.at[chunk], abuf, sa)

    def wait_loads(xbuf, abuf, sx, sa):
        pltpu.make_async_copy(x_hbm.at[pl.ds(0, K)], xbuf, sx).wait()
        pltpu.make_async_copy(av_hbm.at[0], abuf, sa).wait()

    def compute(xbuf, abuf, carry):
        for g in range(K // 16):
            plsc.addupdate_scatter(cnt_f, [abuf[K + g, :]], ones16)

        def slow_rows(g, carry):
            prev = carry[0]
            runs = list(carry[1:])
            for j in range(g * 16, g * 16 + 16):
                addr = abuf[j, :]
                m = plsc.all_reduce_population_count(addr != prev) != 0
                for l in range(NL):
                    plsc.addupdate_scatter(acc_f, [prev + l * 16], runs[l],
                                           mask=m)
                for l in range(NL):
                    xr = xbuf[j, pl.ds(l * 16, 16)]
                    runs[l] = jnp.where(m, xr, runs[l] + xr)
                prev = addr
            return (prev, *runs)

        def fast_rows(g, carry):
            prev = carry[0]
            runs = list(carry[1:])
            a0 = abuf[g * 16, :]
            m = plsc.all_reduce_population_count(a0 != prev) != 0
            for l in range(NL):
                plsc.addupdate_scatter(acc_f, [prev + l * 16], runs[l], mask=m)
            for l in range(NL):
                s = xbuf[g * 16, pl.ds(l * 16, 16)]
                for j in range(1, 16):
                    s = s + xbuf[g * 16 + j, pl.ds(l * 16, 16)]
                runs[l] = jnp.where(m, s, runs[l] + s)
            return (a0, *runs)

        for g in range(K // 16):
            uni = jnp.sum(jnp.where(abuf[g * 16 + 15, :] != abuf[g * 16, :],
                                    1, 0), axis=0) == 0
            carry = lax.cond(uni,
                             lambda c, gg=g: fast_rows(gg, c),
                             lambda c, gg=g: slow_rows(gg, c),
                             carry)
        return carry

    start_loads(c0, xva, ava, sxa, saa)

    def pair_body(t2, carry):
        ca = c0 + 2 * t2
        start_loads(ca + 1, xvb, avb, sxb, sab)
        wait_loads(xva, ava, sxa, saa)
        carry = compute(xva, ava, carry)
        start_loads(ca + 2, xva, ava, sxa, saa)
        wait_loads(xvb, avb, sxb, sab)
        carry = compute(xvb, avb, carry)
        return carry

    final = lax.fori_loop(0, TPW2, pair_body, (prev0, *([zero16] * NL)))
    wait_loads(xva, ava, sxa, saa)
    fprev = final[0]
    for l in range(NL):
        plsc.addupdate_scatter(acc_f, [fprev + l * 16], final[1 + l])

    pltpu.sync_copy(acc_f.at[pl.ds(0, G * C)], acc_out.at[wid])
    pltpu.sync_copy(cnt_f.at[pl.ds(0, G * 16)], cnt_out.at[wid])


def _parta_body(x_ref, b_ref, seg_ref, cnt_ref, seg_acc, cnt_acc):
    i = pl.program_id(0)

    @pl.when(i == 0)
    def _init():
        seg_acc[...] = jnp.zeros_like(seg_acc)
        cnt_acc[...] = jnp.zeros_like(cnt_acc)

    b = b_ref[0, 0, :]
    gi = lax.broadcasted_iota(jnp.int32, (BLK, G), 1)
    oh = jnp.where(gi == b[:, None], 1.0, 0.0).astype(jnp.bfloat16)
    seg_acc[...] += lax.dot_general(
        oh, x_ref[...].astype(jnp.bfloat16), (((0,), (0,)), ((), ())),
        preferred_element_type=jnp.float32)
    cnt_acc[...] += jnp.sum(oh.astype(jnp.float32), axis=0, keepdims=True)

    @pl.when(i == NBLK_A - 1)
    def _finish():
        seg_ref[...] = seg_acc[...]
        cnt_ref[...] = cnt_acc[...]


def _mlp_body(acc_ref, cnt_ref, sega_ref, cnta_ref, w1_ref, a_ref, w2_ref,
              s_ref):
    seg = jnp.sum(acc_ref[...], axis=0) + sega_ref[...]
    cnt_l = jnp.sum(cnt_ref[...], axis=0)
    cnt = jnp.sum(cnt_l, axis=1, keepdims=True) + cnta_ref[...].reshape(G, 1)
    cnt = jnp.maximum(cnt, 1.0)
    x_avg = seg / cnt
    h = lax.dot_general(x_avg, w1_ref[...], (((1,), (1,)), ((), ())),
                        preferred_element_type=jnp.float32)
    a = a_ref[0]
    h = jnp.where(h >= 0, h, a * h)
    h = lax.dot_general(h, w2_ref[...], (((1,), (1,)), ((), ())),
                        preferred_element_type=jnp.float32)
    s_ref[...] = jax.nn.sigmoid(h)


def _scale_body(x_ref, b_ref, s_ref, o_ref):
    b = b_ref[0, 0, :]
    gi = lax.broadcasted_iota(jnp.int32, (BLK, G), 1)
    oh = jnp.where(gi == b[:, None], 1.0, 0.0).astype(jnp.bfloat16)
    se = lax.dot_general(oh, s_ref[...].astype(jnp.bfloat16),
                         (((1,), (0,)), ((), ())),
                         preferred_element_type=jnp.float32)
    o_ref[...] = x_ref[...] * se


def kernel(x, batch, num_graphs, W1, a, W2):
    bi = jnp.minimum(batch, num_graphs - 1).astype(jnp.int32)

    main = bi[:NSC].reshape(NCHUNK, K)
    pad = jnp.full((PCHUNK - NCHUNK, K), TRASH, jnp.int32)
    bidx = jnp.concatenate([main, pad], axis=0)

    lane16 = jnp.arange(16, dtype=jnp.int32)
    av_tab = bidx[:, :, None] * C + lane16[None, None, :]
    ca_tab = (bidx * 16 + jnp.tile(lane16, K // 16)[None, :]).reshape(
        PCHUNK, K // 16, 16)
    aux = jnp.concatenate([av_tab, ca_tab], axis=1)

    zc = jnp.zeros((GPV * C,), jnp.float32)
    z16 = jnp.zeros((GPV * 16,), jnp.float32)

    acc, cnt = _segsum_sc(x, aux, zc, z16)
    acc = acc.reshape(NW, G, C)
    cnt = cnt.reshape(NW, G, 16)

    b3a = bi[NSC:].reshape(NBLK_A, 1, BLK)
    sega, cnta = pl.pallas_call(
        _parta_body,
        grid=(NBLK_A,),
        in_specs=[
            pl.BlockSpec((BLK, C), lambda i: (i + NSC // BLK, 0)),
            pl.BlockSpec((1, 1, BLK), lambda i: (i, 0, 0)),
        ],
        out_specs=[
            pl.BlockSpec((G, C), lambda i: (0, 0)),
            pl.BlockSpec((1, G), lambda i: (0, 0)),
        ],
        out_shape=[
            jax.ShapeDtypeStruct((G, C), jnp.float32),
            jax.ShapeDtypeStruct((1, G), jnp.float32),
        ],
        scratch_shapes=[
            pltpu.VMEM((G, C), jnp.float32),
            pltpu.VMEM((1, G), jnp.float32),
        ],
    )(x, b3a)

    s = pl.pallas_call(
        _mlp_body,
        in_specs=[
            pl.BlockSpec((NW, G, C), lambda: (0, 0, 0)),
            pl.BlockSpec((NW, G, 16), lambda: (0, 0, 0)),
            pl.BlockSpec((G, C), lambda: (0, 0)),
            pl.BlockSpec((1, G), lambda: (0, 0)),
            pl.BlockSpec((H, C), lambda: (0, 0)),
            pl.BlockSpec(memory_space=pltpu.SMEM),
            pl.BlockSpec((C, H), lambda: (0, 0)),
        ],
        out_specs=pl.BlockSpec((G, C), lambda: (0, 0)),
        out_shape=jax.ShapeDtypeStruct((G, C), jnp.float32),
    )(acc, cnt, sega, cnta, W1, a.reshape(1), W2)

    b3 = bi.reshape(NBLK, 1, BLK)
    out = pl.pallas_call(
        _scale_body,
        grid=(NBLK,),
        in_specs=[
            pl.BlockSpec((BLK, C), lambda i: (i, 0)),
            pl.BlockSpec((1, 1, BLK), lambda i: (i, 0, 0)),
            pl.BlockSpec((G, C), lambda i: (0, 0)),
        ],
        out_specs=pl.BlockSpec((BLK, C), lambda i: (i, 0)),
        out_shape=jax.ShapeDtypeStruct((N, C), jnp.float32),
    )(x, b3, s)
    return out
